# baseline (speedup 1.0000x reference)
import functools

import jax
import jax.numpy as jnp
from jax import lax
from jax.experimental import pallas as pl
from jax.experimental.pallas import tpu as pltpu
from jax.experimental.pallas import tpu_sc as plsc

N_NEWS = 10000
N_USERS = 10000
N_EDGES = 320000
D = 128
DEGW = 16

NC = 2
NS = 16
NW = NC * NS
EPT = N_EDGES // NW
CHUNK = 40
NCHUNK = EPT // CHUNK
NU_PAD = 10240
ROWS_PT = NU_PAD // NS


def _sc_body(news_hbm, row_hbm, col_hbm, agr_out, deg_out,
             row_v, col_v, gbuf, ones_v, zdeg, agr_sh, deg_sh,
             sem0, sem1, sem_d):
    c = lax.axis_index("c")
    s = lax.axis_index("s")
    wid = s * NC + c

    zeros16 = jnp.zeros((16,), jnp.float32)
    ones16 = jnp.ones((16,), jnp.float32)

    def zfill(i, _):
        r = i // 8
        col8 = (i % 8) * 16
        gbuf[r, pl.ds(col8, 16)] = zeros16
        return 0
    lax.fori_loop(0, 2 * CHUNK * (D // 16), zfill, 0)

    def zdfill(i, _):
        zdeg[i, pl.ds(0, 16)] = zeros16
        ones_v[i, pl.ds(0, 16)] = ones16
        return 0
    lax.fori_loop(0, CHUNK, zdfill, 0)

    base = s * ROWS_PT
    for k in range(ROWS_PT // (2 * CHUNK)):
        pltpu.sync_copy(gbuf,
                        agr_sh.at[pl.ds(base + k * 2 * CHUNK, 2 * CHUNK)])
        pltpu.sync_copy(zdeg, deg_sh.at[pl.ds(base + k * 2 * CHUNK, CHUNK)])
        pltpu.sync_copy(
            zdeg, deg_sh.at[pl.ds(base + k * 2 * CHUNK + CHUNK, CHUNK)])

    pltpu.sync_copy(row_hbm.at[wid], row_v)
    pltpu.sync_copy(col_hbm.at[wid], col_v)

    half = (gbuf.at[pl.ds(0, CHUNK)], gbuf.at[pl.ds(CHUNK, CHUNK)])
    sem_g = (sem0, sem1)

    def wait_rows(sem, dst):
        pltpu.make_async_copy(news_hbm.at[pl.ds(0, CHUNK)], dst, sem).wait()

    pltpu.async_copy(news_hbm.at[row_v.at[0]], half[0], sem0)

    plsc.subcore_barrier()

    def stage(j, b):
        wait_rows(sem_g[b], half[b])

        @pl.when(j + 1 < NCHUNK)
        def _():
            pltpu.async_copy(news_hbm.at[row_v.at[j + 1]], half[1 - b],
                             sem_g[1 - b])

        pltpu.async_copy(ones_v, deg_sh.at[col_v.at[j]], sem_d, add=True)
        pltpu.sync_copy(half[b], agr_sh.at[col_v.at[j]], add=True)

    def step(t, _):
        stage(2 * t, 0)
        stage(2 * t + 1, 1)
        return 0
    lax.fori_loop(0, NCHUNK // 2, step, 0)

    def drain(j, _):
        pltpu.make_async_copy(deg_out.at[0].at[pl.ds(0, CHUNK)], zdeg,
                              sem_d).wait()
        return 0
    lax.fori_loop(0, NCHUNK, drain, 0)

    plsc.subcore_barrier()

    for k in range(ROWS_PT // (2 * CHUNK)):
        sl = pl.ds(base + k * 2 * CHUNK, 2 * CHUNK)
        pltpu.sync_copy(agr_sh.at[sl], agr_out.at[c].at[sl])
        pltpu.sync_copy(deg_sh.at[sl], deg_out.at[c].at[sl])


_sc_call = functools.partial(
    pl.kernel,
    out_type=[
        jax.ShapeDtypeStruct((NC, NU_PAD, D), jnp.float32),
        jax.ShapeDtypeStruct((NC, NU_PAD, DEGW), jnp.float32),
    ],
    mesh=plsc.VectorSubcoreMesh(core_axis_name="c", subcore_axis_name="s",
                                num_cores=NC, num_subcores=NS),
    scratch_types=[
        pltpu.VMEM((NCHUNK, CHUNK), jnp.int32),
        pltpu.VMEM((NCHUNK, CHUNK), jnp.int32),
        pltpu.VMEM((2 * CHUNK, D), jnp.float32),
        pltpu.VMEM((CHUNK, DEGW), jnp.float32),
        pltpu.VMEM((CHUNK, DEGW), jnp.float32),
        pltpu.VMEM_SHARED((NU_PAD, D), jnp.float32),
        pltpu.VMEM_SHARED((NU_PAD, DEGW), jnp.float32),
        pltpu.SemaphoreType.DMA,
        pltpu.SemaphoreType.DMA,
        pltpu.SemaphoreType.DMA,
    ],
    compiler_params=pltpu.CompilerParams(use_tc_tiling_on_sc=False),
)(_sc_body)


BLK = 1024


def _mlp_body(user_ref, agrp_ref, degp_ref, w1u_ref, w1a_ref, b1_ref,
              w2_ref, b2_ref, out_ref):
    agr = agrp_ref[0] + agrp_ref[1]
    deg = degp_ref[0, :, 0:1] + degp_ref[1, :, 0:1]
    agr = agr / (deg + 1e-8)
    h = jnp.tanh(
        jnp.dot(user_ref[...], w1u_ref[...], preferred_element_type=jnp.float32)
        + jnp.dot(agr, w1a_ref[...], preferred_element_type=jnp.float32)
        + b1_ref[...])
    out_ref[...] = (
        jnp.dot(h, w2_ref[...], preferred_element_type=jnp.float32)
        + b2_ref[...])


def _mlp_call(user_feats, agr_p, deg_p, w1u, w1a, b1, w2, b2):
    grid = (NU_PAD // BLK,)
    return pl.pallas_call(
        _mlp_body,
        grid=grid,
        in_specs=[
            pl.BlockSpec((BLK, D), lambda i: (i, 0)),
            pl.BlockSpec((NC, BLK, D), lambda i: (0, i, 0)),
            pl.BlockSpec((NC, BLK, DEGW), lambda i: (0, i, 0)),
            pl.BlockSpec((D, D), lambda i: (0, 0)),
            pl.BlockSpec((D, D), lambda i: (0, 0)),
            pl.BlockSpec((1, D), lambda i: (0, 0)),
            pl.BlockSpec((D, D), lambda i: (0, 0)),
            pl.BlockSpec((1, D), lambda i: (0, 0)),
        ],
        out_specs=pl.BlockSpec((BLK, D), lambda i: (i, 0)),
        out_shape=jax.ShapeDtypeStruct((NU_PAD, D), jnp.float32),
    )(user_feats, agr_p, deg_p, w1u, w1a, b1, w2, b2)


def kernel(news_feats, user_feats, edge_index, W1, b1, W2, b2):
    row = edge_index[0].astype(jnp.int32).reshape(NW, NCHUNK, CHUNK)
    col = edge_index[1].astype(jnp.int32).reshape(NW, NCHUNK, CHUNK)
    agr_p, deg_p = _sc_call(news_feats, row, col)
    w1u = W1[:, :D].T
    w1a = W1[:, D:].T
    w2 = W2.T
    user_pad = jnp.pad(user_feats, ((0, NU_PAD - N_USERS), (0, 0)))
    out = _mlp_call(user_pad, agr_p, deg_p, w1u, w1a,
                    b1.reshape(1, D), w2, b2.reshape(1, D))
    return out[:N_USERS]


# speedup vs baseline: 1.0481x; 1.0481x over previous
import functools

import jax
import jax.numpy as jnp
from jax import lax
from jax.experimental import pallas as pl
from jax.experimental.pallas import tpu as pltpu
from jax.experimental.pallas import tpu_sc as plsc

N_NEWS = 10000
N_USERS = 10000
N_EDGES = 320000
D = 128
DEGW = 16

NC = 2
NS = 16
NW = NC * NS
EPT = N_EDGES // NW
CHUNK = 80
NCHUNK = EPT // CHUNK
NU_PAD = 10240
ROWS_PT = NU_PAD // NS


def _sc_body(news_hbm, row_hbm, col_hbm, agr_out, deg_out,
             row_v, col_v, gbuf, ones_v, zdeg, agr_sh, deg_sh, sem, sem_d):
    c = lax.axis_index("c")
    s = lax.axis_index("s")
    wid = s * NC + c

    zeros16 = jnp.zeros((16,), jnp.float32)
    ones16 = jnp.ones((16,), jnp.float32)

    def zfill(i, _):
        r = i // 8
        col8 = (i % 8) * 16
        gbuf[r, pl.ds(col8, 16)] = zeros16
        return 0
    lax.fori_loop(0, CHUNK * (D // 16), zfill, 0)

    def zdfill(i, _):
        zdeg[i, pl.ds(0, 16)] = zeros16
        ones_v[i, pl.ds(0, 16)] = ones16
        return 0
    lax.fori_loop(0, CHUNK, zdfill, 0)

    base = s * ROWS_PT
    for k in range(ROWS_PT // CHUNK):
        pltpu.sync_copy(gbuf, agr_sh.at[pl.ds(base + k * CHUNK, CHUNK)])
        pltpu.sync_copy(zdeg, deg_sh.at[pl.ds(base + k * CHUNK, CHUNK)])

    pltpu.sync_copy(row_hbm.at[wid], row_v)
    pltpu.sync_copy(col_hbm.at[wid], col_v)

    plsc.subcore_barrier()

    def step(j, _):
        pltpu.async_copy(news_hbm.at[row_v.at[j]], gbuf, sem).wait()
        pltpu.async_copy(ones_v, deg_sh.at[col_v.at[j]], sem_d, add=True)
        pltpu.sync_copy(gbuf, agr_sh.at[col_v.at[j]], add=True)
        return 0
    lax.fori_loop(0, NCHUNK, step, 0)

    def drain(j, _):
        pltpu.make_async_copy(deg_out.at[0].at[pl.ds(0, CHUNK)], zdeg,
                              sem_d).wait()
        return 0
    lax.fori_loop(0, NCHUNK, drain, 0)

    plsc.subcore_barrier()

    for k in range(ROWS_PT // CHUNK):
        sl = pl.ds(base + k * CHUNK, CHUNK)
        pltpu.sync_copy(agr_sh.at[sl], agr_out.at[c].at[sl])
        pltpu.sync_copy(deg_sh.at[sl], deg_out.at[c].at[sl])


_sc_call = functools.partial(
    pl.kernel,
    out_type=[
        jax.ShapeDtypeStruct((NC, NU_PAD, D), jnp.float32),
        jax.ShapeDtypeStruct((NC, NU_PAD, DEGW), jnp.float32),
    ],
    mesh=plsc.VectorSubcoreMesh(core_axis_name="c", subcore_axis_name="s",
                                num_cores=NC, num_subcores=NS),
    scratch_types=[
        pltpu.VMEM((NCHUNK, CHUNK), jnp.int32),
        pltpu.VMEM((NCHUNK, CHUNK), jnp.int32),
        pltpu.VMEM((CHUNK, D), jnp.float32),
        pltpu.VMEM((CHUNK, DEGW), jnp.float32),
        pltpu.VMEM((CHUNK, DEGW), jnp.float32),
        pltpu.VMEM_SHARED((NU_PAD, D), jnp.float32),
        pltpu.VMEM_SHARED((NU_PAD, DEGW), jnp.float32),
        pltpu.SemaphoreType.DMA,
        pltpu.SemaphoreType.DMA,
    ],
    compiler_params=pltpu.CompilerParams(use_tc_tiling_on_sc=False),
)(_sc_body)


BLK = 400


def _mlp_body(user_ref, agrp_ref, degp_ref, w1u_ref, w1a_ref, b1_ref,
              w2_ref, b2_ref, out_ref):
    agr = agrp_ref[0] + agrp_ref[1]
    deg = degp_ref[0, :, 0:1] + degp_ref[1, :, 0:1]
    agr = agr / (deg + 1e-8)
    h = jnp.tanh(
        jnp.dot(user_ref[...], w1u_ref[...], preferred_element_type=jnp.float32)
        + jnp.dot(agr, w1a_ref[...], preferred_element_type=jnp.float32)
        + b1_ref[...])
    out_ref[...] = (
        jnp.dot(h, w2_ref[...], preferred_element_type=jnp.float32)
        + b2_ref[...])


def _mlp_call(user_feats, agr_p, deg_p, w1u, w1a, b1, w2, b2):
    grid = (N_USERS // BLK,)
    return pl.pallas_call(
        _mlp_body,
        grid=grid,
        in_specs=[
            pl.BlockSpec((BLK, D), lambda i: (i, 0)),
            pl.BlockSpec((NC, BLK, D), lambda i: (0, i, 0)),
            pl.BlockSpec((NC, BLK, DEGW), lambda i: (0, i, 0)),
            pl.BlockSpec((D, D), lambda i: (0, 0)),
            pl.BlockSpec((D, D), lambda i: (0, 0)),
            pl.BlockSpec((1, D), lambda i: (0, 0)),
            pl.BlockSpec((D, D), lambda i: (0, 0)),
            pl.BlockSpec((1, D), lambda i: (0, 0)),
        ],
        out_specs=pl.BlockSpec((BLK, D), lambda i: (i, 0)),
        out_shape=jax.ShapeDtypeStruct((N_USERS, D), jnp.float32),
    )(user_feats, agr_p, deg_p, w1u, w1a, b1, w2, b2)


def kernel(news_feats, user_feats, edge_index, W1, b1, W2, b2):
    row = edge_index[0].astype(jnp.int32).reshape(NW, NCHUNK, CHUNK)
    col = edge_index[1].astype(jnp.int32).reshape(NW, NCHUNK, CHUNK)
    agr_p, deg_p = _sc_call(news_feats, row, col)
    w1u = W1[:, :D].T
    w1a = W1[:, D:].T
    w2 = W2.T
    return _mlp_call(user_feats, agr_p, deg_p, w1u, w1a,
                     b1.reshape(1, D), w2, b2.reshape(1, D))


# speedup vs baseline: 1.0794x; 1.0298x over previous
import functools

import jax
import jax.numpy as jnp
from jax import lax
from jax.experimental import pallas as pl
from jax.experimental.pallas import tpu as pltpu
from jax.experimental.pallas import tpu_sc as plsc

N_NEWS = 10000
N_USERS = 10000
N_EDGES = 320000
D = 128
DEGW = 16

NC = 2
NS = 16
NW = NC * NS
EPT = N_EDGES // NW
CHUNK = 80
NCHUNK = EPT // CHUNK
NU_PAD = 10240
ROWS_PT = NU_PAD // NS


def _sc_body(news_hbm, row_hbm, col_hbm, agr_out, deg_out,
             row_v, col_v, gbuf, ones_v, zdeg, agr_sh, deg_sh, sem, sem_d):
    c = lax.axis_index("c")
    s = lax.axis_index("s")
    wid = s * NC + c

    zeros16 = jnp.zeros((16,), jnp.float32)
    ones16 = jnp.ones((16,), jnp.float32)

    def zfill(i, _):
        r = i // 8
        col8 = (i % 8) * 16
        gbuf[r, pl.ds(col8, 16)] = zeros16
        return 0
    lax.fori_loop(0, CHUNK * (D // 16), zfill, 0)

    def zdfill(i, _):
        zdeg[i, pl.ds(0, 16)] = zeros16
        ones_v[i, pl.ds(0, 16)] = ones16
        return 0
    lax.fori_loop(0, CHUNK, zdfill, 0)

    base = s * ROWS_PT
    for k in range(ROWS_PT // CHUNK):
        pltpu.sync_copy(gbuf, agr_sh.at[pl.ds(base + k * CHUNK, CHUNK)])
        pltpu.sync_copy(zdeg, deg_sh.at[pl.ds(base + k * CHUNK, CHUNK)])

    pltpu.sync_copy(row_hbm.at[wid], row_v)
    pltpu.sync_copy(col_hbm.at[wid], col_v)

    plsc.subcore_barrier()

    def step(j, _):
        pltpu.async_copy(news_hbm.at[row_v.at[j]], gbuf, sem).wait()
        pltpu.async_copy(ones_v, deg_sh.at[col_v.at[j]], sem_d, add=True)
        pltpu.sync_copy(gbuf, agr_sh.at[col_v.at[j]], add=True)
        return 0
    lax.fori_loop(0, NCHUNK, step, 0)

    def drain(j, _):
        pltpu.make_async_copy(deg_out.at[0].at[pl.ds(0, CHUNK)], zdeg,
                              sem_d).wait()
        return 0
    lax.fori_loop(0, NCHUNK, drain, 0)

    plsc.subcore_barrier()

    for k in range(ROWS_PT // CHUNK):
        sl = pl.ds(base + k * CHUNK, CHUNK)
        pltpu.sync_copy(agr_sh.at[sl], agr_out.at[c].at[sl])
        pltpu.sync_copy(deg_sh.at[sl], deg_out.at[c].at[sl])


_sc_call = functools.partial(
    pl.kernel,
    out_type=[
        jax.ShapeDtypeStruct((NC, NU_PAD, D), jnp.float32),
        jax.ShapeDtypeStruct((NC, NU_PAD, DEGW), jnp.float32),
    ],
    mesh=plsc.VectorSubcoreMesh(core_axis_name="c", subcore_axis_name="s",
                                num_cores=NC, num_subcores=NS),
    scratch_types=[
        pltpu.VMEM((NCHUNK, CHUNK), jnp.int32),
        pltpu.VMEM((NCHUNK, CHUNK), jnp.int32),
        pltpu.VMEM((CHUNK, D), jnp.float32),
        pltpu.VMEM((CHUNK, DEGW), jnp.float32),
        pltpu.VMEM((CHUNK, DEGW), jnp.float32),
        pltpu.VMEM_SHARED((NU_PAD, D), jnp.float32),
        pltpu.VMEM_SHARED((NU_PAD, DEGW), jnp.float32),
        pltpu.SemaphoreType.DMA,
        pltpu.SemaphoreType.DMA,
    ],
    compiler_params=pltpu.CompilerParams(use_tc_tiling_on_sc=False),
)(_sc_body)


BLK = 1000


def _mlp_body(user_ref, agrp_ref, degp_ref, w1u_ref, w1a_ref, b1_ref,
              w2_ref, b2_ref, out_ref):
    agr = agrp_ref[0] + agrp_ref[1]
    deg = degp_ref[0, :, 0:1] + degp_ref[1, :, 0:1]
    agr = agr / (deg + 1e-8)
    h = jnp.tanh(
        jnp.dot(user_ref[...], w1u_ref[...], preferred_element_type=jnp.float32)
        + jnp.dot(agr, w1a_ref[...], preferred_element_type=jnp.float32)
        + b1_ref[...])
    out_ref[...] = (
        jnp.dot(h, w2_ref[...], preferred_element_type=jnp.float32)
        + b2_ref[...])


def _mlp_call(user_feats, agr_p, deg_p, w1u, w1a, b1, w2, b2):
    grid = (N_USERS // BLK,)
    return pl.pallas_call(
        _mlp_body,
        grid=grid,
        in_specs=[
            pl.BlockSpec((BLK, D), lambda i: (i, 0)),
            pl.BlockSpec((NC, BLK, D), lambda i: (0, i, 0)),
            pl.BlockSpec((NC, BLK, DEGW), lambda i: (0, i, 0)),
            pl.BlockSpec((D, D), lambda i: (0, 0)),
            pl.BlockSpec((D, D), lambda i: (0, 0)),
            pl.BlockSpec((1, D), lambda i: (0, 0)),
            pl.BlockSpec((D, D), lambda i: (0, 0)),
            pl.BlockSpec((1, D), lambda i: (0, 0)),
        ],
        out_specs=pl.BlockSpec((BLK, D), lambda i: (i, 0)),
        out_shape=jax.ShapeDtypeStruct((N_USERS, D), jnp.float32),
    )(user_feats, agr_p, deg_p, w1u, w1a, b1, w2, b2)


def kernel(news_feats, user_feats, edge_index, W1, b1, W2, b2):
    row = edge_index[0].astype(jnp.int32).reshape(NW, NCHUNK, CHUNK)
    col = edge_index[1].astype(jnp.int32).reshape(NW, NCHUNK, CHUNK)
    agr_p, deg_p = _sc_call(news_feats, row, col)
    w1u = W1[:, :D].T
    w1a = W1[:, D:].T
    w2 = W2.T
    return _mlp_call(user_feats, agr_p, deg_p, w1u, w1a,
                     b1.reshape(1, D), w2, b2.reshape(1, D))


# speedup vs baseline: 1.1018x; 1.0207x over previous
import functools

import jax
import jax.numpy as jnp
from jax import lax
from jax.experimental import pallas as pl
from jax.experimental.pallas import tpu as pltpu
from jax.experimental.pallas import tpu_sc as plsc

N_NEWS = 10000
N_USERS = 10000
N_EDGES = 320000
D = 128
DEGW = 16

NC = 2
NS = 16
NW = NC * NS
EPT = N_EDGES // NW
CHUNK = 80
NCHUNK = EPT // CHUNK
NU_PAD = 10240
ROWS_PT = NU_PAD // NS


def _sc_body(news_hbm, row_hbm, col_hbm, agr_out, deg_out,
             row_v, col_v, gbuf, ones_v, zdeg, agr_sh, deg_sh, sem, sem_d):
    c = lax.axis_index("c")
    s = lax.axis_index("s")
    wid = s * NC + c

    zeros16 = jnp.zeros((16,), jnp.float32)
    ones16 = jnp.ones((16,), jnp.float32)

    def zfill(i, _):
        r = i // 8
        col8 = (i % 8) * 16
        gbuf[r, pl.ds(col8, 16)] = zeros16
        return 0
    lax.fori_loop(0, CHUNK * (D // 16), zfill, 0)

    def zdfill(i, _):
        zdeg[i, pl.ds(0, 16)] = zeros16
        ones_v[i, pl.ds(0, 16)] = ones16
        return 0
    lax.fori_loop(0, CHUNK, zdfill, 0)

    base = s * ROWS_PT
    for k in range(ROWS_PT // CHUNK):
        pltpu.sync_copy(gbuf, agr_sh.at[pl.ds(base + k * CHUNK, CHUNK)])
        pltpu.sync_copy(zdeg, deg_sh.at[pl.ds(base + k * CHUNK, CHUNK)])

    pltpu.sync_copy(row_hbm.at[wid], row_v)
    pltpu.sync_copy(col_hbm.at[wid], col_v)

    plsc.subcore_barrier()

    def step(j, _):
        g = pltpu.async_copy(news_hbm.at[row_v.at[j]], gbuf, sem)
        pltpu.async_copy(ones_v, deg_sh.at[col_v.at[j]], sem_d, add=True)
        g.wait()
        pltpu.sync_copy(gbuf, agr_sh.at[col_v.at[j]], add=True)
        return 0
    lax.fori_loop(0, NCHUNK, step, 0)

    def drain(j, _):
        pltpu.make_async_copy(deg_out.at[0].at[pl.ds(0, CHUNK)], zdeg,
                              sem_d).wait()
        return 0
    lax.fori_loop(0, NCHUNK, drain, 0)

    plsc.subcore_barrier()

    for k in range(ROWS_PT // CHUNK):
        sl = pl.ds(base + k * CHUNK, CHUNK)
        pltpu.sync_copy(agr_sh.at[sl], agr_out.at[c].at[sl])
        pltpu.sync_copy(deg_sh.at[sl], deg_out.at[c].at[sl])


_sc_call = functools.partial(
    pl.kernel,
    out_type=[
        jax.ShapeDtypeStruct((NC, NU_PAD, D), jnp.float32),
        jax.ShapeDtypeStruct((NC, NU_PAD, DEGW), jnp.float32),
    ],
    mesh=plsc.VectorSubcoreMesh(core_axis_name="c", subcore_axis_name="s",
                                num_cores=NC, num_subcores=NS),
    scratch_types=[
        pltpu.VMEM((NCHUNK, CHUNK), jnp.int32),
        pltpu.VMEM((NCHUNK, CHUNK), jnp.int32),
        pltpu.VMEM((CHUNK, D), jnp.float32),
        pltpu.VMEM((CHUNK, DEGW), jnp.float32),
        pltpu.VMEM((CHUNK, DEGW), jnp.float32),
        pltpu.VMEM_SHARED((NU_PAD, D), jnp.float32),
        pltpu.VMEM_SHARED((NU_PAD, DEGW), jnp.float32),
        pltpu.SemaphoreType.DMA,
        pltpu.SemaphoreType.DMA,
    ],
    compiler_params=pltpu.CompilerParams(use_tc_tiling_on_sc=False),
)(_sc_body)


BLK = 1000


def _mlp_body(user_ref, agrp_ref, degp_ref, w1u_ref, w1a_ref, b1_ref,
              w2_ref, b2_ref, out_ref):
    agr = agrp_ref[0] + agrp_ref[1]
    deg = degp_ref[0, :, 0:1] + degp_ref[1, :, 0:1]
    agr = agr / (deg + 1e-8)
    h = jnp.tanh(
        jnp.dot(user_ref[...], w1u_ref[...], preferred_element_type=jnp.float32)
        + jnp.dot(agr, w1a_ref[...], preferred_element_type=jnp.float32)
        + b1_ref[...])
    out_ref[...] = (
        jnp.dot(h, w2_ref[...], preferred_element_type=jnp.float32)
        + b2_ref[...])


def _mlp_call(user_feats, agr_p, deg_p, w1u, w1a, b1, w2, b2):
    grid = (N_USERS // BLK,)
    return pl.pallas_call(
        _mlp_body,
        grid=grid,
        in_specs=[
            pl.BlockSpec((BLK, D), lambda i: (i, 0)),
            pl.BlockSpec((NC, BLK, D), lambda i: (0, i, 0)),
            pl.BlockSpec((NC, BLK, DEGW), lambda i: (0, i, 0)),
            pl.BlockSpec((D, D), lambda i: (0, 0)),
            pl.BlockSpec((D, D), lambda i: (0, 0)),
            pl.BlockSpec((1, D), lambda i: (0, 0)),
            pl.BlockSpec((D, D), lambda i: (0, 0)),
            pl.BlockSpec((1, D), lambda i: (0, 0)),
        ],
        out_specs=pl.BlockSpec((BLK, D), lambda i: (i, 0)),
        out_shape=jax.ShapeDtypeStruct((N_USERS, D), jnp.float32),
    )(user_feats, agr_p, deg_p, w1u, w1a, b1, w2, b2)


def kernel(news_feats, user_feats, edge_index, W1, b1, W2, b2):
    row = edge_index[0].astype(jnp.int32).reshape(NW, NCHUNK, CHUNK)
    col = edge_index[1].astype(jnp.int32).reshape(NW, NCHUNK, CHUNK)
    agr_p, deg_p = _sc_call(news_feats, row, col)
    w1u = W1[:, :D].T
    w1a = W1[:, D:].T
    w2 = W2.T
    return _mlp_call(user_feats, agr_p, deg_p, w1u, w1a,
                     b1.reshape(1, D), w2, b2.reshape(1, D))
